# baseline (speedup 1.0000x reference)
import functools

import jax
import jax.numpy as jnp
from jax import lax
from jax.experimental import pallas as pl
from jax.experimental.pallas import tpu as pltpu
from jax.experimental.pallas import tpu_sc as plsc

N = 10000
E = 320000
D = 128
H = 64

NPAD = 10240
ROWB = 512
NBLK = NPAD // ROWB
NC = 2
NS = 16
NW = NC * NS
CH = 128
NCHUNK = 79
NCH0 = 69
NCH1 = 89
NCHM = 89
NBUF = 2
EPAD = NW * NCHUNK * CH
E0 = NS * NCH0 * CH
RA = 2 * NPAD
RPSA = RA // NS
RC = 2 * NPAD + 128
DUMP = 2 * NPAD
RPSC = RC // NS
NZR = 8
NPADZ = NPAD + ROWB


def _sc_mesh():
  return plsc.VectorSubcoreMesh(
      core_axis_name="c", subcore_axis_name="s",
      num_cores=NC, num_subcores=NS)


def _make_agg():
  out_type = (
      jax.ShapeDtypeStruct((NC, RA, 64), jnp.float32),
      jax.ShapeDtypeStruct((NC, RA, 64), jnp.float32),
  )
  scratch = [
      pltpu.VMEM((NCHM, CH), jnp.int32),
      pltpu.VMEM((NCHM, CH), jnp.int32),
  ] + [pltpu.VMEM((CH, 64), jnp.float32) for _ in range(NBUF)] + [
      pltpu.VMEM((CH, 64), jnp.float32),
      pltpu.VMEM_SHARED((RA, 64), jnp.float32),
  ] + [pltpu.SemaphoreType.DMA for _ in range(NBUF)]

  def body(fa, fb, gidx_h, sidx_h, zc64_h, oa, ob, gidx, sidx, *rest):
    bufs = rest[:NBUF]
    z64 = rest[NBUF]
    acc = rest[NBUF + 1]
    sems = rest[NBUF + 2:]
    cid = lax.axis_index("c")
    sid = lax.axis_index("s")
    wid = cid * NS + sid
    nch = jnp.where(cid == 0, NCH0, NCH1)
    base = sid * RPSA

    pltpu.sync_copy(gidx_h.at[wid], gidx)
    pltpu.sync_copy(sidx_h.at[wid], sidx)
    pltpu.sync_copy(zc64_h, z64)

    def zero_acc():
      for t in range(10):
        pltpu.sync_copy(z64, acc.at[pl.ds(base + t * CH, CH)])

    zero_acc()
    plsc.subcore_barrier()

    def run_phase(f_hbm, out_ref):
      def scat(k, buf):
        pltpu.sync_copy(buf, acc.at[sidx.at[k]], add=True)

      def gs(k, buf, sem):
        pltpu.async_copy(f_hbm.at[gidx.at[k]], buf, sem)

      def gw(buf, sem):
        pltpu.make_async_copy(f_hbm.at[gidx.at[0]], buf, sem).wait()

      buf0, buf1 = bufs[0], bufs[1]
      sem0, sem1 = sems[0], sems[1]
      gs(0, buf0, sem0)

      def loop(k, carry):
        a = 2 * k
        gs(a + 1, buf1, sem1)
        gw(buf0, sem0)
        scat(a, buf0)
        gs(a + 2, buf0, sem0)
        gw(buf1, sem1)
        scat(a + 1, buf1)
        return carry

      lax.fori_loop(0, (nch - 1) // 2, loop, 0)
      gw(buf0, sem0)
      scat(nch - 1, buf0)
      plsc.subcore_barrier()
      pltpu.sync_copy(acc.at[pl.ds(base, RPSA)],
                      out_ref.at[cid, pl.ds(base, RPSA)])

    run_phase(fa, oa)
    plsc.subcore_barrier()
    zero_acc()
    plsc.subcore_barrier()
    run_phase(fb, ob)

  return pl.kernel(body, out_type=out_type, mesh=_sc_mesh(),
                   scratch_types=scratch,
                   compiler_params=pltpu.CompilerParams(
                       use_tc_tiling_on_sc=False))


def _make_counts():
  scratch = [
      pltpu.VMEM((NCHUNK, CH), jnp.int32),
      pltpu.VMEM((CH, 16), jnp.float32),
      pltpu.VMEM((CH, 16), jnp.float32),
      pltpu.VMEM_SHARED((RC, 16), jnp.float32),
  ]

  def body(sidx_h, oc16_h, zc16_h, oc, sidx, ones16, z16, cacc):
    cid = lax.axis_index("c")
    sid = lax.axis_index("s")
    wid = sid * NC + cid
    base = sid * RPSC

    pltpu.sync_copy(sidx_h.at[wid], sidx)
    pltpu.sync_copy(oc16_h, ones16)
    pltpu.sync_copy(zc16_h, z16)
    for t in range(10):
      pltpu.sync_copy(z16, cacc.at[pl.ds(base + t * CH, CH)])
    pltpu.sync_copy(z16.at[pl.ds(0, 8)], cacc.at[pl.ds(base + 10 * CH, 8)])
    plsc.subcore_barrier()

    def loop(k, carry):
      pltpu.sync_copy(ones16, cacc.at[sidx.at[k]], add=True)
      return carry

    lax.fori_loop(0, NCHUNK, loop, 0)
    plsc.subcore_barrier()
    pltpu.sync_copy(cacc.at[pl.ds(base, RPSC)],
                    oc.at[cid, pl.ds(base, RPSC)])

  return pl.kernel(body,
                   out_type=jax.ShapeDtypeStruct((NC, RC, 16), jnp.float32),
                   mesh=_sc_mesh(), scratch_types=scratch,
                   compiler_params=pltpu.CompilerParams(
                       use_tc_tiling_on_sc=False))


def _t1_body(x_ref, w_ref, b_ref, oa_ref, ob_ref):
  live = pl.program_id(0) < NBLK
  h = jnp.dot(x_ref[...], w_ref[...],
              preferred_element_type=jnp.float32) + b_ref[...]
  oa_ref[...] = jnp.where(live, h[:, :H], 0.0)
  ob_ref[...] = jnp.where(live, h[:, H:], 0.0)


def _t2_body(pap, pan, pbp, pbn, cp_ref, cn_ref, h0a, h0b,
             wp1, wn1, bp1, bn1, zp_ref, zn_ref):
  live = pl.program_id(0) < NBLK
  cp = jnp.maximum(cp_ref[0, :, 0:1] + cp_ref[1, :, 0:1], 1.0)
  cn = jnp.maximum(cn_ref[0, :, 0:1] + cn_ref[1, :, 0:1], 1.0)
  a = h0a[...]
  b = h0b[...]
  dot = functools.partial(jnp.dot, preferred_element_type=jnp.float32)
  xp_cat = jnp.concatenate(
      [(pap[0] + pap[1]) / cp, (pbp[0] + pbp[1]) / cp, a, b], axis=1)
  xn_cat = jnp.concatenate(
      [(pan[0] + pan[1]) / cn, (pbn[0] + pbn[1]) / cn, a, b], axis=1)
  hp = dot(xp_cat, wp1[...]) + bp1[...]
  hn = dot(xn_cat, wn1[...]) + bn1[...]
  zp_ref[...] = jnp.where(live, jnp.tanh(hp), 0.0)
  zn_ref[...] = jnp.where(live, jnp.tanh(hn), 0.0)


def _t3_body(qap, qan, qbp, qbn, cp_ref, cn_ref, zp_ref, zn_ref,
             wp2, wn2, bp2, bn2, ww, bw, wm1, bm1, g1, be1,
             wm2, bm2, g2, be2, wm3t, bm3, z_ref, prob_ref):
  cp = jnp.maximum(cp_ref[0, :, 0:1] + cp_ref[1, :, 0:1], 1.0)
  cn = jnp.maximum(cn_ref[0, :, 0:1] + cn_ref[1, :, 0:1], 1.0)
  zp = zp_ref[...]
  zn = zn_ref[...]
  dot = functools.partial(jnp.dot, preferred_element_type=jnp.float32)
  xp_cat = jnp.concatenate(
      [(qap[0] + qap[1]) / cp, (qbn[0] + qbn[1]) / cn, zp], axis=1)
  xn_cat = jnp.concatenate(
      [(qbp[0] + qbp[1]) / cp, (qan[0] + qan[1]) / cn, zn], axis=1)
  hp = dot(xp_cat, wp2[...]) + bp2[...]
  hn = dot(xn_cat, wn2[...]) + bn2[...]
  z2 = jnp.concatenate([jnp.tanh(hp), jnp.tanh(hn)], axis=1)
  z = jnp.tanh(dot(z2, ww[...]) + bw[...])
  z_ref[...] = z
  rs = 1.0 / jnp.sqrt(1.0 + 1e-5)
  h1 = jax.nn.relu(g1[...] * (dot(z, wm1[...]) + bm1[...]) * rs + be1[...])
  h2 = jax.nn.relu(g2[...] * (dot(h1, wm2[...]) + bm2[...]) * rs + be2[...])
  logit = jnp.sum(h2 * wm3t[...], axis=1, keepdims=True) + bm3[0, 0]
  prob_ref[...] = jax.nn.sigmoid(logit)


def _row_spec(shape):
  return pl.BlockSpec((ROWB,) + shape[1:], lambda i: (i,) + (0,) * (len(shape) - 1))


def _row_spec_cl(shape):
  return pl.BlockSpec((ROWB,) + shape[1:],
                      lambda i: (jnp.minimum(i, NBLK - 1),)
                      + (0,) * (len(shape) - 1))


def _full_spec(shape):
  return pl.BlockSpec(shape, lambda i: (0,) * len(shape))


def _part_spec(width, neg):
  off = NBLK if neg else 0
  return pl.BlockSpec(
      (NC, ROWB, width),
      lambda i, off=off: (0, jnp.minimum(off + i, 2 * NBLK - 1), 0))


def kernel(x, edge_index, W_init, b_init, Wp1, bp1, Wn1, bn1, Wp2, bp2,
           Wn2, bn2, Ww, bw, Wm1, bm1, g1, be1, Wm2, bm2, g2, be2, Wm3, bm3):
  f32 = jnp.float32
  src = edge_index[:, 0].astype(jnp.int32)
  dst = edge_index[:, 1].astype(jnp.int32)
  sign = edge_index[:, 2]
  sidx = dst + NPAD * (sign < 0).astype(jnp.int32)
  npad_e = EPAD - E
  pad_ar = jnp.arange(npad_e, dtype=jnp.int32)
  gidx_p = jnp.concatenate([src, NPAD + pad_ar % NZR])
  sidx_p = jnp.concatenate([sidx, pad_ar % RA])
  cidx_p = jnp.concatenate([sidx, DUMP + pad_ar % 128])

  def _skew(a):
    c0 = jnp.pad(a[:E0].reshape(NS, NCH0, CH),
                 ((0, 0), (0, NCHM - NCH0), (0, 0)))
    c1 = a[E0:].reshape(NS, NCH1, CH)
    return jnp.concatenate([c0, c1], axis=0)

  gidx3 = _skew(gidx_p)
  sidx3 = _skew(sidx_p)
  cidx3 = cidx_p.reshape(NW, NCHUNK, CH)

  xp = jnp.pad(x, ((0, NPAD - N), (0, 0)))
  z64 = jnp.zeros((CH, 64), f32)
  o16 = jnp.ones((CH, 16), f32)
  z16 = jnp.zeros((CH, 16), f32)

  h0a, h0b = pl.pallas_call(
      _t1_body,
      grid=(NBLK + 1,),
      in_specs=[_row_spec_cl((NPAD, H)), _full_spec((H, D)),
                _full_spec((1, D))],
      out_specs=[_row_spec((NPADZ, H)), _row_spec((NPADZ, H))],
      out_shape=[jax.ShapeDtypeStruct((NPADZ, H), f32)] * 2,
  )(xp, W_init, b_init.reshape(1, D))

  cnt = _make_counts()(cidx3, o16, z16)
  z64 = z64 + 0.0 * cnt[0, 0, 0]
  pa, pb = _make_agg()(h0a, h0b, gidx3, sidx3, z64)

  wspec = [_full_spec((4 * H, H)), _full_spec((4 * H, H)),
           _full_spec((1, H)), _full_spec((1, H))]
  zp, zn = pl.pallas_call(
      _t2_body,
      grid=(NBLK + 1,),
      in_specs=[_part_spec(64, False), _part_spec(64, True),
                _part_spec(64, False), _part_spec(64, True),
                _part_spec(16, False), _part_spec(16, True),
                _row_spec((NPADZ, H)), _row_spec((NPADZ, H))] + wspec,
      out_specs=[_row_spec((NPADZ, H)), _row_spec((NPADZ, H))],
      out_shape=[jax.ShapeDtypeStruct((NPADZ, H), f32)] * 2,
  )(pa, pa, pb, pb, cnt, cnt, h0a, h0b,
    Wp1, Wn1, bp1.reshape(1, H), bn1.reshape(1, H))

  qa, qb = _make_agg()(zp, zn, gidx3, sidx3, z64)

  w3spec = [_full_spec((3 * H, H)), _full_spec((3 * H, H)),
            _full_spec((1, H)), _full_spec((1, H)),
            _full_spec((D, D)), _full_spec((1, D)),
            _full_spec((D, D)), _full_spec((1, D)),
            _full_spec((1, D)), _full_spec((1, D)),
            _full_spec((D, D)), _full_spec((1, D)),
            _full_spec((1, D)), _full_spec((1, D)),
            _full_spec((1, D)), _full_spec((1, 1))]
  z, prob = pl.pallas_call(
      _t3_body,
      grid=(NBLK,),
      in_specs=[_part_spec(64, False), _part_spec(64, True),
                _part_spec(64, False), _part_spec(64, True),
                _part_spec(16, False), _part_spec(16, True),
                _row_spec((NPADZ, H)), _row_spec((NPADZ, H))] + w3spec,
      out_specs=[_row_spec((N, D)), _row_spec((N, 1))],
      out_shape=[jax.ShapeDtypeStruct((N, D), f32),
                 jax.ShapeDtypeStruct((N, 1), f32)],
  )(qa, qa, qb, qb, cnt, cnt, zp, zn,
    Wp2, Wn2, bp2.reshape(1, H), bn2.reshape(1, H),
    Ww, bw.reshape(1, D), Wm1, bm1.reshape(1, D),
    g1.reshape(1, D), be1.reshape(1, D), Wm2, bm2.reshape(1, D),
    g2.reshape(1, D), be2.reshape(1, D),
    Wm3.reshape(1, D), bm3.reshape(1, 1))

  return (z, prob)


# speedup vs baseline: 1.0676x; 1.0676x over previous
import functools

import jax
import jax.numpy as jnp
from jax import lax
from jax.experimental import pallas as pl
from jax.experimental.pallas import tpu as pltpu
from jax.experimental.pallas import tpu_sc as plsc

N = 10000
E = 320000
D = 128
H = 64

NPAD = 10240
ROWB = 512
NBLK = NPAD // ROWB
NC = 2
NS = 16
NW = NC * NS
CH = 128
NCHUNK = 79
NCH0 = 89
NCH1 = 69
NCHM = 89
NBUF = 2
EPAD = NW * NCHUNK * CH
E0 = NS * NCH0 * CH
RA = 2 * NPAD
RPSA = RA // NS
RC = 2 * NPAD + 128
DUMP = 2 * NPAD
RPSC = RC // NS
NZR = 8
NPADZ = NPAD + ROWB


def _sc_mesh():
  return plsc.VectorSubcoreMesh(
      core_axis_name="c", subcore_axis_name="s",
      num_cores=NC, num_subcores=NS)


def _make_agg():
  out_type = (
      jax.ShapeDtypeStruct((NC, RA, 64), jnp.float32),
      jax.ShapeDtypeStruct((NC, RA, 64), jnp.float32),
  )
  scratch = [
      pltpu.VMEM((NCHM, CH), jnp.int32),
      pltpu.VMEM((NCHM, CH), jnp.int32),
  ] + [pltpu.VMEM((CH, 64), jnp.float32) for _ in range(NBUF)] + [
      pltpu.VMEM((CH, 64), jnp.float32),
      pltpu.VMEM_SHARED((RA, 64), jnp.float32),
  ] + [pltpu.SemaphoreType.DMA for _ in range(NBUF)]

  def body(fa, fb, gidx_h, sidx_h, zc64_h, oa, ob, gidx, sidx, *rest):
    bufs = rest[:NBUF]
    z64 = rest[NBUF]
    acc = rest[NBUF + 1]
    sems = rest[NBUF + 2:]
    cid = lax.axis_index("c")
    sid = lax.axis_index("s")
    wid = cid * NS + sid
    nch = jnp.where(cid == 0, NCH0, NCH1)
    base = sid * RPSA

    pltpu.sync_copy(gidx_h.at[wid], gidx)
    pltpu.sync_copy(sidx_h.at[wid], sidx)
    pltpu.sync_copy(zc64_h, z64)

    def zero_acc():
      for t in range(10):
        pltpu.sync_copy(z64, acc.at[pl.ds(base + t * CH, CH)])

    zero_acc()
    plsc.subcore_barrier()

    def run_phase(f_hbm, out_ref):
      def scat(k, buf):
        pltpu.sync_copy(buf, acc.at[sidx.at[k]], add=True)

      def gs(k, buf, sem):
        pltpu.async_copy(f_hbm.at[gidx.at[k]], buf, sem)

      def gw(buf, sem):
        pltpu.make_async_copy(f_hbm.at[gidx.at[0]], buf, sem).wait()

      buf0, buf1 = bufs[0], bufs[1]
      sem0, sem1 = sems[0], sems[1]
      gs(0, buf0, sem0)

      def loop(k, carry):
        a = 2 * k
        gs(a + 1, buf1, sem1)
        gw(buf0, sem0)
        scat(a, buf0)
        gs(a + 2, buf0, sem0)
        gw(buf1, sem1)
        scat(a + 1, buf1)
        return carry

      lax.fori_loop(0, (nch - 1) // 2, loop, 0)
      gw(buf0, sem0)
      scat(nch - 1, buf0)
      plsc.subcore_barrier()
      pltpu.sync_copy(acc.at[pl.ds(base, RPSA)],
                      out_ref.at[cid, pl.ds(base, RPSA)])

    run_phase(fa, oa)
    plsc.subcore_barrier()
    zero_acc()
    plsc.subcore_barrier()
    run_phase(fb, ob)

  return pl.kernel(body, out_type=out_type, mesh=_sc_mesh(),
                   scratch_types=scratch,
                   compiler_params=pltpu.CompilerParams(
                       use_tc_tiling_on_sc=False))


def _make_counts():
  scratch = [
      pltpu.VMEM((NCHUNK, CH), jnp.int32),
      pltpu.VMEM((CH, 16), jnp.float32),
      pltpu.VMEM((CH, 16), jnp.float32),
      pltpu.VMEM_SHARED((RC, 16), jnp.float32),
  ]

  def body(sidx_h, oc16_h, zc16_h, oc, sidx, ones16, z16, cacc):
    cid = lax.axis_index("c")
    sid = lax.axis_index("s")
    wid = sid * NC + cid
    base = sid * RPSC

    pltpu.sync_copy(sidx_h.at[wid], sidx)
    pltpu.sync_copy(oc16_h, ones16)
    pltpu.sync_copy(zc16_h, z16)
    for t in range(10):
      pltpu.sync_copy(z16, cacc.at[pl.ds(base + t * CH, CH)])
    pltpu.sync_copy(z16.at[pl.ds(0, 8)], cacc.at[pl.ds(base + 10 * CH, 8)])
    plsc.subcore_barrier()

    def loop(k, carry):
      pltpu.sync_copy(ones16, cacc.at[sidx.at[k]], add=True)
      return carry

    lax.fori_loop(0, NCHUNK, loop, 0)
    plsc.subcore_barrier()
    pltpu.sync_copy(cacc.at[pl.ds(base, RPSC)],
                    oc.at[cid, pl.ds(base, RPSC)])

  return pl.kernel(body,
                   out_type=jax.ShapeDtypeStruct((NC, RC, 16), jnp.float32),
                   mesh=_sc_mesh(), scratch_types=scratch,
                   compiler_params=pltpu.CompilerParams(
                       use_tc_tiling_on_sc=False))


def _t1_body(x_ref, w_ref, b_ref, oa_ref, ob_ref):
  live = pl.program_id(0) < NBLK
  h = jnp.dot(x_ref[...], w_ref[...],
              preferred_element_type=jnp.float32) + b_ref[...]
  oa_ref[...] = jnp.where(live, h[:, :H], 0.0)
  ob_ref[...] = jnp.where(live, h[:, H:], 0.0)


def _t2_body(pap, pan, pbp, pbn, cp_ref, cn_ref, h0a, h0b,
             wp1, wn1, bp1, bn1, zp_ref, zn_ref):
  live = pl.program_id(0) < NBLK
  cp = jnp.maximum(cp_ref[0, :, 0:1] + cp_ref[1, :, 0:1], 1.0)
  cn = jnp.maximum(cn_ref[0, :, 0:1] + cn_ref[1, :, 0:1], 1.0)
  a = h0a[...]
  b = h0b[...]
  dot = functools.partial(jnp.dot, preferred_element_type=jnp.float32)
  xp_cat = jnp.concatenate(
      [(pap[0] + pap[1]) / cp, (pbp[0] + pbp[1]) / cp, a, b], axis=1)
  xn_cat = jnp.concatenate(
      [(pan[0] + pan[1]) / cn, (pbn[0] + pbn[1]) / cn, a, b], axis=1)
  hp = dot(xp_cat, wp1[...]) + bp1[...]
  hn = dot(xn_cat, wn1[...]) + bn1[...]
  zp_ref[...] = jnp.where(live, jnp.tanh(hp), 0.0)
  zn_ref[...] = jnp.where(live, jnp.tanh(hn), 0.0)


def _t3_body(qap, qan, qbp, qbn, cp_ref, cn_ref, zp_ref, zn_ref,
             wp2, wn2, bp2, bn2, ww, bw, wm1, bm1, g1, be1,
             wm2, bm2, g2, be2, wm3t, bm3, z_ref, prob_ref):
  cp = jnp.maximum(cp_ref[0, :, 0:1] + cp_ref[1, :, 0:1], 1.0)
  cn = jnp.maximum(cn_ref[0, :, 0:1] + cn_ref[1, :, 0:1], 1.0)
  zp = zp_ref[...]
  zn = zn_ref[...]
  dot = functools.partial(jnp.dot, preferred_element_type=jnp.float32)
  xp_cat = jnp.concatenate(
      [(qap[0] + qap[1]) / cp, (qbn[0] + qbn[1]) / cn, zp], axis=1)
  xn_cat = jnp.concatenate(
      [(qbp[0] + qbp[1]) / cp, (qan[0] + qan[1]) / cn, zn], axis=1)
  hp = dot(xp_cat, wp2[...]) + bp2[...]
  hn = dot(xn_cat, wn2[...]) + bn2[...]
  z2 = jnp.concatenate([jnp.tanh(hp), jnp.tanh(hn)], axis=1)
  z = jnp.tanh(dot(z2, ww[...]) + bw[...])
  z_ref[...] = z
  rs = 1.0 / jnp.sqrt(1.0 + 1e-5)
  h1 = jax.nn.relu(g1[...] * (dot(z, wm1[...]) + bm1[...]) * rs + be1[...])
  h2 = jax.nn.relu(g2[...] * (dot(h1, wm2[...]) + bm2[...]) * rs + be2[...])
  logit = jnp.sum(h2 * wm3t[...], axis=1, keepdims=True) + bm3[0, 0]
  prob_ref[...] = jax.nn.sigmoid(logit)


def _row_spec(shape):
  return pl.BlockSpec((ROWB,) + shape[1:], lambda i: (i,) + (0,) * (len(shape) - 1))


def _row_spec_cl(shape):
  return pl.BlockSpec((ROWB,) + shape[1:],
                      lambda i: (jnp.minimum(i, NBLK - 1),)
                      + (0,) * (len(shape) - 1))


def _full_spec(shape):
  return pl.BlockSpec(shape, lambda i: (0,) * len(shape))


def _part_spec(width, neg):
  off = NBLK if neg else 0
  return pl.BlockSpec(
      (NC, ROWB, width),
      lambda i, off=off: (0, jnp.minimum(off + i, 2 * NBLK - 1), 0))


def kernel(x, edge_index, W_init, b_init, Wp1, bp1, Wn1, bn1, Wp2, bp2,
           Wn2, bn2, Ww, bw, Wm1, bm1, g1, be1, Wm2, bm2, g2, be2, Wm3, bm3):
  f32 = jnp.float32
  src = edge_index[:, 0].astype(jnp.int32)
  dst = edge_index[:, 1].astype(jnp.int32)
  sign = edge_index[:, 2]
  sidx = dst + NPAD * (sign < 0).astype(jnp.int32)
  npad_e = EPAD - E
  pad_ar = jnp.arange(npad_e, dtype=jnp.int32)
  gidx_p = jnp.concatenate([src, NPAD + pad_ar % NZR])
  sidx_p = jnp.concatenate([sidx, pad_ar % RA])
  cidx_p = jnp.concatenate([sidx, DUMP + pad_ar % 128])

  def _skew(a):
    c0 = jnp.pad(a[:E0].reshape(NS, NCH0, CH),
                 ((0, 0), (0, NCHM - NCH0), (0, 0)))
    c1 = jnp.pad(a[E0:].reshape(NS, NCH1, CH),
                 ((0, 0), (0, NCHM - NCH1), (0, 0)))
    return jnp.concatenate([c0, c1], axis=0)

  gidx3 = _skew(gidx_p)
  sidx3 = _skew(sidx_p)
  cidx3 = cidx_p.reshape(NW, NCHUNK, CH)

  xp = jnp.pad(x, ((0, NPAD - N), (0, 0)))
  z64 = jnp.zeros((CH, 64), f32)
  o16 = jnp.ones((CH, 16), f32)
  z16 = jnp.zeros((CH, 16), f32)

  h0a, h0b = pl.pallas_call(
      _t1_body,
      grid=(NBLK + 1,),
      in_specs=[_row_spec_cl((NPAD, H)), _full_spec((H, D)),
                _full_spec((1, D))],
      out_specs=[_row_spec((NPADZ, H)), _row_spec((NPADZ, H))],
      out_shape=[jax.ShapeDtypeStruct((NPADZ, H), f32)] * 2,
  )(xp, W_init, b_init.reshape(1, D))

  cnt = _make_counts()(cidx3, o16, z16)
  z64 = z64 + 0.0 * cnt[0, 0, 0]
  pa, pb = _make_agg()(h0a, h0b, gidx3, sidx3, z64)

  wspec = [_full_spec((4 * H, H)), _full_spec((4 * H, H)),
           _full_spec((1, H)), _full_spec((1, H))]
  zp, zn = pl.pallas_call(
      _t2_body,
      grid=(NBLK + 1,),
      in_specs=[_part_spec(64, False), _part_spec(64, True),
                _part_spec(64, False), _part_spec(64, True),
                _part_spec(16, False), _part_spec(16, True),
                _row_spec((NPADZ, H)), _row_spec((NPADZ, H))] + wspec,
      out_specs=[_row_spec((NPADZ, H)), _row_spec((NPADZ, H))],
      out_shape=[jax.ShapeDtypeStruct((NPADZ, H), f32)] * 2,
  )(pa, pa, pb, pb, cnt, cnt, h0a, h0b,
    Wp1, Wn1, bp1.reshape(1, H), bn1.reshape(1, H))

  qa, qb = _make_agg()(zp, zn, gidx3, sidx3, z64)

  w3spec = [_full_spec((3 * H, H)), _full_spec((3 * H, H)),
            _full_spec((1, H)), _full_spec((1, H)),
            _full_spec((D, D)), _full_spec((1, D)),
            _full_spec((D, D)), _full_spec((1, D)),
            _full_spec((1, D)), _full_spec((1, D)),
            _full_spec((D, D)), _full_spec((1, D)),
            _full_spec((1, D)), _full_spec((1, D)),
            _full_spec((1, D)), _full_spec((1, 1))]
  z, prob = pl.pallas_call(
      _t3_body,
      grid=(NBLK,),
      in_specs=[_part_spec(64, False), _part_spec(64, True),
                _part_spec(64, False), _part_spec(64, True),
                _part_spec(16, False), _part_spec(16, True),
                _row_spec((NPADZ, H)), _row_spec((NPADZ, H))] + w3spec,
      out_specs=[_row_spec((N, D)), _row_spec((N, 1))],
      out_shape=[jax.ShapeDtypeStruct((N, D), f32),
                 jax.ShapeDtypeStruct((N, 1), f32)],
  )(qa, qa, qb, qb, cnt, cnt, zp, zn,
    Wp2, Wn2, bp2.reshape(1, H), bn2.reshape(1, H),
    Ww, bw.reshape(1, D), Wm1, bm1.reshape(1, D),
    g1.reshape(1, D), be1.reshape(1, D), Wm2, bm2.reshape(1, D),
    g2.reshape(1, D), be2.reshape(1, D),
    Wm3.reshape(1, D), bm3.reshape(1, 1))

  return (z, prob)
